# baseline (speedup 1.0000x reference)
import functools

import jax
import jax.numpy as jnp
from jax import lax
from jax.experimental import pallas as pl
from jax.experimental.pallas import tpu as pltpu
from jax.experimental.pallas import tpu_sc as plsc

BINS = 256
NB2 = BINS * BINS
H = 512
W = 512
NCH = 24
CHUNK = 32
NCHUNK = H // CHUNK
ROWW = W + 16
NVEC = W // 16

D0 = -0.5 * 3.14158202
D1 = 0.5 * 5.16714272
D2 = -0.5 * 2.54189839
D3 = 0.5 * 0.55463468

CLIP_HI = 254.99998474121094


def _sc_body(x_hbm, out_hbm, hist, inbuf, w0a, k0a, sem0, sem1):
    info = plsc.get_sparse_core_info()
    nc = info.num_cores
    wid = lax.axis_index("s") * nc + lax.axis_index("c")
    ch = jnp.minimum(wid, NCH - 1)

    copy0 = pltpu.async_copy(x_hbm.at[ch, pl.ds(0, CHUNK)], inbuf.at[0], sem0)

    zf = jnp.zeros((16,), jnp.float32)

    def zero_body(i, _):
        for j in range(16):
            hist[pl.ds(i * 256 + j * 16, 16)] = zf
        return 0

    lax.fori_loop(0, NB2 // 256, zero_body, 0)
    zi = jnp.zeros((16,), jnp.int32)
    w0a[pl.ds(W, 16)] = zf
    k0a[pl.ds(W, 16)] = zi
    mask15 = lax.iota(jnp.int32, 16) < (16 - 1)

    def _px_block(xs):
        xs = [jnp.minimum(x, CLIP_HI) for x in xs]
        ks = [x.astype(jnp.int32) for x in xs]
        kf = [kk.astype(jnp.float32) for kk in ks]
        us = [(x - g) - 0.5 for x, g in zip(xs, kf)]
        zs = [u * u for u in us]
        ps = [D2 + z * D3 for z in zs]
        ps = [D1 + z * p for z, p in zip(zs, ps)]
        ps = [D0 + z * p for z, p in zip(zs, ps)]
        w0s = [0.5 + u * p for u, p in zip(us, ps)]
        return w0s, ks

    def _pair_block(vs, last_masked=False):
        bs = [v * 16 for v in vs]
        w0ls = [w0a[pl.ds(b, 16)] for b in bs]
        k0ls = [k0a[pl.ds(b, 16)] for b in bs]
        w0rs = [w0a[pl.ds(b + 1, 16)] for b in bs]
        k0rs = [k0a[pl.ds(b + 1, 16)] for b in bs]
        f00s = [kl * BINS + kr for kl, kr in zip(k0ls, k0rs)]
        p00s = [wl * wr for wl, wr in zip(w0ls, w0rs)]
        p01s = [wl - p for wl, p in zip(w0ls, p00s)]
        p10s = [wr - p for wr, p in zip(w0rs, p00s)]
        w1ls = [1.0 - wl for wl in w0ls]
        p11s = [w1 - t for w1, t in zip(w1ls, p10s)]
        n = len(vs)
        for j in range(n):
            f00 = f00s[j]
            m = mask15 if (last_masked and j == n - 1) else None
            plsc.addupdate_scatter(hist, [f00], p00s[j])
            plsc.addupdate_scatter(hist, [f00 + 1], p01s[j], mask=m)
            plsc.addupdate_scatter(hist, [f00 + BINS], p10s[j])
            plsc.addupdate_scatter(hist, [f00 + (BINS + 1)], p11s[j], mask=m)

    def row_body(buf):
        def body(r, _):
            for k in range(NVEC // 8):
                base = k * 128
                xs = [inbuf[buf, r, pl.ds(base + 16 * j, 16)] for j in range(8)]
                w0s, ks = _px_block(xs)
                for j in range(8):
                    w0a[pl.ds(base + 16 * j, 16)] = w0s[j]
                for j in range(8):
                    k0a[pl.ds(base + 16 * j, 16)] = ks[j]

            for k in range(NVEC // 8):
                _pair_block(
                    [k * 8 + j for j in range(8)],
                    last_masked=(k == NVEC // 8 - 1),
                )
            return 0

        lax.fori_loop(0, CHUNK, body, 0)

    sems = (sem0, sem1)
    copies = [None] * NCHUNK
    copies[0] = copy0
    for g in range(NCHUNK):
        if g + 1 < NCHUNK:
            copies[g + 1] = pltpu.async_copy(
                x_hbm.at[ch, pl.ds((g + 1) * CHUNK, CHUNK)],
                inbuf.at[(g + 1) % 2],
                sems[(g + 1) % 2],
            )
        copies[g].wait()
        row_body(g % 2)

    def max_body(i, accs):
        return tuple(
            jnp.maximum(accs[j], hist[pl.ds(i * 256 + j * 16, 16)]) for j in range(16)
        )

    accs = lax.fori_loop(0, NB2 // 256, max_body, (zf,) * 16)
    acc = accs[0]
    for j in range(1, 16):
        acc = jnp.maximum(acc, accs[j])
    inv = 1.0 / jnp.broadcast_to(jnp.max(acc), (16,))

    def scale_body(i, _):
        for j in range(16):
            hist[pl.ds(i * 256 + j * 16, 16)] = hist[pl.ds(i * 256 + j * 16, 16)] * inv
        return 0

    lax.fori_loop(0, NB2 // 256, scale_body, 0)

    @pl.when(wid < NCH)
    def _():
        pltpu.sync_copy(hist, out_hbm.at[ch])


def kernel(X):
    B, C, h, w = X.shape
    x = X.reshape(B * C, h, w)
    mesh = plsc.VectorSubcoreMesh(core_axis_name="c", subcore_axis_name="s")
    hist_fn = pl.kernel(
        _sc_body,
        out_type=jax.ShapeDtypeStruct((NCH, NB2), jnp.float32),
        mesh=mesh,
        compiler_params=pltpu.CompilerParams(needs_layout_passes=False),
        scratch_types=[
            pltpu.VMEM((NB2,), jnp.float32),
            pltpu.VMEM((2, CHUNK, W), jnp.float32),
            pltpu.VMEM((ROWW,), jnp.float32),
            pltpu.VMEM((ROWW,), jnp.int32),
            pltpu.SemaphoreType.DMA,
            pltpu.SemaphoreType.DMA,
        ],
    )
    out = hist_fn(x)
    return out.reshape(B, C, BINS, BINS)


# speedup vs baseline: 1.0266x; 1.0266x over previous
import functools

import jax
import jax.numpy as jnp
from jax import lax
from jax.experimental import pallas as pl
from jax.experimental.pallas import tpu as pltpu
from jax.experimental.pallas import tpu_sc as plsc

BINS = 256
NB2 = BINS * BINS
H = 512
W = 512
NCH = 24
CHUNK = 32
NCHUNK = H // CHUNK
ROWW = W + 16
NVEC = W // 16

D0 = -0.5 * 3.1413147117971043
D1 = 0.5 * 5.147662186517426
D2 = -0.5 * 2.3339087474671096

CLIP_HI = 254.99998474121094


def _sc_body(x_hbm, out_hbm, hist, inbuf, w0a, k0a, sem0, sem1):
    info = plsc.get_sparse_core_info()
    nc = info.num_cores
    wid = lax.axis_index("s") * nc + lax.axis_index("c")
    ch = jnp.minimum(wid, NCH - 1)

    copy0 = pltpu.async_copy(x_hbm.at[ch, pl.ds(0, CHUNK)], inbuf.at[0], sem0)

    zf = jnp.zeros((16,), jnp.float32)

    def zero_body(i, _):
        for j in range(16):
            hist[pl.ds(i * 256 + j * 16, 16)] = zf
        return 0

    lax.fori_loop(0, NB2 // 256, zero_body, 0)
    zi = jnp.zeros((16,), jnp.int32)
    w0a[pl.ds(W, 16)] = zf
    k0a[pl.ds(W, 16)] = zi
    mask15 = lax.iota(jnp.int32, 16) < (16 - 1)

    def _px_block(xs):
        xs = [jnp.minimum(x, CLIP_HI) for x in xs]
        ks = [x.astype(jnp.int32) for x in xs]
        kf = [kk.astype(jnp.float32) for kk in ks]
        us = [(x - g) - 0.5 for x, g in zip(xs, kf)]
        zs = [u * u for u in us]
        ps = [D1 + z * D2 for z in zs]
        ps = [D0 + z * p for z, p in zip(zs, ps)]
        w0s = [0.5 + u * p for u, p in zip(us, ps)]
        return w0s, ks

    def _pair_block(vs, last_masked=False):
        bs = [v * 16 for v in vs]
        w0ls = [w0a[pl.ds(b, 16)] for b in bs]
        k0ls = [k0a[pl.ds(b, 16)] for b in bs]
        w0rs = [w0a[pl.ds(b + 1, 16)] for b in bs]
        k0rs = [k0a[pl.ds(b + 1, 16)] for b in bs]
        f00s = [kl * BINS + kr for kl, kr in zip(k0ls, k0rs)]
        p00s = [wl * wr for wl, wr in zip(w0ls, w0rs)]
        p01s = [wl - p for wl, p in zip(w0ls, p00s)]
        p10s = [wr - p for wr, p in zip(w0rs, p00s)]
        w1ls = [1.0 - wl for wl in w0ls]
        p11s = [w1 - t for w1, t in zip(w1ls, p10s)]
        n = len(vs)
        for j in range(n):
            f00 = f00s[j]
            m = mask15 if (last_masked and j == n - 1) else None
            plsc.addupdate_scatter(hist, [f00], p00s[j])
            plsc.addupdate_scatter(hist, [f00 + 1], p01s[j], mask=m)
            plsc.addupdate_scatter(hist, [f00 + BINS], p10s[j])
            plsc.addupdate_scatter(hist, [f00 + (BINS + 1)], p11s[j], mask=m)

    def row_body(buf):
        def body(r, _):
            for k in range(NVEC // 8):
                base = k * 128
                xs = [inbuf[buf, r, pl.ds(base + 16 * j, 16)] for j in range(8)]
                w0s, ks = _px_block(xs)
                for j in range(8):
                    w0a[pl.ds(base + 16 * j, 16)] = w0s[j]
                for j in range(8):
                    k0a[pl.ds(base + 16 * j, 16)] = ks[j]

            for k in range(NVEC // 8):
                _pair_block(
                    [k * 8 + j for j in range(8)],
                    last_masked=(k == NVEC // 8 - 1),
                )
            return 0

        lax.fori_loop(0, CHUNK, body, 0)

    sems = (sem0, sem1)
    copies = [None] * NCHUNK
    copies[0] = copy0
    for g in range(NCHUNK):
        if g + 1 < NCHUNK:
            copies[g + 1] = pltpu.async_copy(
                x_hbm.at[ch, pl.ds((g + 1) * CHUNK, CHUNK)],
                inbuf.at[(g + 1) % 2],
                sems[(g + 1) % 2],
            )
        copies[g].wait()
        row_body(g % 2)

    def max_body(i, accs):
        return tuple(
            jnp.maximum(accs[j], hist[pl.ds(i * 256 + j * 16, 16)]) for j in range(16)
        )

    accs = lax.fori_loop(0, NB2 // 256, max_body, (zf,) * 16)
    acc = accs[0]
    for j in range(1, 16):
        acc = jnp.maximum(acc, accs[j])
    inv = 1.0 / jnp.broadcast_to(jnp.max(acc), (16,))

    def scale_body(i, _):
        for j in range(16):
            hist[pl.ds(i * 256 + j * 16, 16)] = hist[pl.ds(i * 256 + j * 16, 16)] * inv
        return 0

    lax.fori_loop(0, NB2 // 256, scale_body, 0)

    @pl.when(wid < NCH)
    def _():
        pltpu.sync_copy(hist, out_hbm.at[ch])


def kernel(X):
    B, C, h, w = X.shape
    x = X.reshape(B * C, h, w)
    mesh = plsc.VectorSubcoreMesh(core_axis_name="c", subcore_axis_name="s")
    hist_fn = pl.kernel(
        _sc_body,
        out_type=jax.ShapeDtypeStruct((NCH, NB2), jnp.float32),
        mesh=mesh,
        compiler_params=pltpu.CompilerParams(needs_layout_passes=False),
        scratch_types=[
            pltpu.VMEM((NB2,), jnp.float32),
            pltpu.VMEM((2, CHUNK, W), jnp.float32),
            pltpu.VMEM((ROWW,), jnp.float32),
            pltpu.VMEM((ROWW,), jnp.int32),
            pltpu.SemaphoreType.DMA,
            pltpu.SemaphoreType.DMA,
        ],
    )
    out = hist_fn(x)
    return out.reshape(B, C, BINS, BINS)
